# baseline (speedup 1.0000x reference)
import functools

import jax
import jax.numpy as jnp
from jax import lax
from jax.experimental import pallas as pl
from jax.experimental.pallas import tpu as pltpu
from jax.experimental.pallas import tpu_sc as plsc

N = 10000
E = 320000
NF_DIM = 128
HNF = 128
HEF = 64
YW = 128

NC_ = 2
NS_ = 16
NW = NC_ * NS_
EW = E // NW
EH = E // 2
EWH = EH // NW
GCH = 200
NGCH = EWH // GCH
NGPAIR = (NGCH - 1) // 2
SCH = 80
SG = 1
NSCH = EW // (SCH * SG)
NSPAIR = (NSCH - 1) // 2
NSTRIPE = 624
NTAIL = N - NS_ * NSTRIPE


def _m8(x):
    return pl.multiple_of(x, 8)


def _m16(x):
    return pl.multiple_of(x, 16)

_mesh = plsc.VectorSubcoreMesh(core_axis_name="c", subcore_axis_name="s")


def _make_gather(off):
    @functools.partial(
        pl.kernel,
        out_type=jax.ShapeDtypeStruct((EH, 128), jnp.float32),
        mesh=_mesh,
        scratch_types=(
            pltpu.VMEM((EWH,), jnp.int32),
            pltpu.VMEM((EWH,), jnp.int32),
            pltpu.VMEM((2, GCH, 128), jnp.float32),
            pltpu.VMEM((2, GCH, 128), jnp.float32),
            pltpu.SemaphoreType.DMA,
            pltpu.SemaphoreType.DMA,
            pltpu.SemaphoreType.DMA,
            pltpu.SemaphoreType.DMA,
            pltpu.SemaphoreType.DMA,
            pltpu.SemaphoreType.DMA,
        ),
    )
    def _gather(pq_hbm, src_hbm, dst_hbm, gsd_hbm,
                sidx, didx, rows_s, rows_d, gs0, gs1, gd0, gd1, w0, w1):
        c = lax.axis_index("c")
        s = lax.axis_index("s")
        w = c * NS_ + s
        base0 = w * EWH
        pltpu.sync_copy(src_hbm.at[pl.ds(off + base0, EWH)], sidx)
        pltpu.sync_copy(dst_hbm.at[pl.ds(off + base0, EWH)], didx)

        gssem = (gs0, gs1)
        gdsem = (gd0, gd1)
        wsem = (w0, w1)

        def gath(ch, slot):
            pltpu.async_copy(
                pq_hbm.at[sidx.at[pl.ds(ch * GCH, GCH)]], rows_s.at[slot],
                gssem[slot])
            pltpu.async_copy(
                pq_hbm.at[didx.at[pl.ds(ch * GCH, GCH)]], rows_d.at[slot],
                gdsem[slot])

        def wout(ch, slot):
            return pltpu.async_copy(
                rows_s.at[slot],
                gsd_hbm.at[pl.ds(_m8(base0 + ch * GCH), GCH)], wsem[slot])

        def wait_g(slot):
            pltpu.make_async_copy(
                pq_hbm.at[sidx.at[pl.ds(0, GCH)]], rows_s.at[slot],
                gssem[slot]).wait()
            pltpu.make_async_copy(
                pq_hbm.at[didx.at[pl.ds(0, GCH)]], rows_d.at[slot],
                gdsem[slot]).wait()

        def wait_w(slot):
            pltpu.make_async_copy(
                rows_s.at[slot], gsd_hbm.at[pl.ds(base0, GCH)],
                wsem[slot]).wait()

        def add_halves(slot):
            @pl.loop(0, GCH, unroll=2)
            def _add(r):
                for k in range(HEF // 16):
                    sl = pl.ds(k * 16, 16)
                    sr = pl.ds(HEF + k * 16, 16)
                    rows_s[slot, r, sl] = (rows_s[slot, r, sl]
                                           + rows_d[slot, r, sr])

        gath(0, 0)

        @pl.loop(0, NGPAIR)
        def _pair(ii):
            i0 = 2 * ii

            @pl.when(ii > 0)
            def _():
                wait_w(1)

            gath(i0 + 1, 1)
            wait_g(0)
            add_halves(0)
            wout(i0, 0)
            wait_w(0)
            gath(i0 + 2, 0)
            wait_g(1)
            add_halves(1)
            wout(i0 + 1, 1)

        wait_w(1)
        wait_g(0)
        add_halves(0)
        wout(NGCH - 1, 0)
        wait_w(0)

    return _gather


_sc_gather_a = _make_gather(0)
_sc_gather_b = _make_gather(EH)


@functools.partial(
    pl.kernel,
    out_type=jax.ShapeDtypeStruct((NC_ * N, YW), jnp.float32),
    mesh=_mesh,
    scratch_types=(
        pltpu.VMEM((2, SCH, YW), jnp.float32),
        [pltpu.VMEM((SCH,), jnp.int32) for _ in range(2)],
        pltpu.VMEM_SHARED((N, YW), jnp.float32),
        pltpu.SemaphoreType.DMA,
        pltpu.SemaphoreType.DMA,
        pltpu.SemaphoreType.DMA,
        pltpu.SemaphoreType.DMA,
        pltpu.SemaphoreType.DMA,
        pltpu.SemaphoreType.DMA,
    ),
)
def _sc_scatter(ya_hbm, yb_hbm, dst_hbm, zer_hbm, z_hbm,
                ybuf, idxb, table, l0, l1, a0, a1, x0, x1):
    c = lax.axis_index("c")
    s = lax.axis_index("s")
    w = c * NS_ + s
    base0 = w * EW
    ybase0 = s * EW
    stripe = pl.ds(_m8(s * NSTRIPE), NSTRIPE)
    pltpu.sync_copy(zer_hbm.at[stripe], table.at[stripe])

    @pl.when(s == 0)
    def _ztail():
        tail = pl.ds(NS_ * NSTRIPE, NTAIL)
        pltpu.sync_copy(zer_hbm.at[tail], table.at[tail])

    lsem = (l0, l1)
    asem = (a0, a1)
    xsem = (x0, x1)

    def load(ch, slot):
        @pl.when(c == 0)
        def _la():
            pltpu.async_copy(
                ya_hbm.at[pl.ds(_m8(ybase0 + ch * SCH), SCH)],
                ybuf.at[slot], lsem[slot])

        @pl.when(c == 1)
        def _lb():
            pltpu.async_copy(
                yb_hbm.at[pl.ds(_m8(ybase0 + ch * SCH), SCH)],
                ybuf.at[slot], lsem[slot])

        pltpu.async_copy(
            dst_hbm.at[pl.ds(_m8(base0 + ch * SCH), SCH)],
            idxb[slot], xsem[slot])

    def scat(ch, slot):
        pltpu.async_copy(
            ybuf.at[slot], table.at[idxb[slot]], asem[slot], add=True)

    def wait_l(slot):
        pltpu.make_async_copy(
            ya_hbm.at[pl.ds(ybase0, SCH)], ybuf.at[slot], lsem[slot]).wait()
        pltpu.make_async_copy(
            dst_hbm.at[pl.ds(base0, SCH)], idxb[slot], xsem[slot]).wait()

    def wait_a(slot):
        pltpu.make_async_copy(
            ybuf.at[slot], table.at[idxb[slot]], asem[slot]).wait()

    load(0, 0)
    plsc.subcore_barrier()

    @pl.loop(0, NSPAIR)
    def _pair(ii):
        i0 = 2 * ii

        @pl.when(ii > 0)
        def _():
            wait_a(1)

        load(i0 + 1, 1)
        wait_l(0)
        scat(i0, 0)
        wait_a(0)
        load(i0 + 2, 0)
        wait_l(1)
        scat(i0 + 1, 1)

    wait_a(1)
    wait_l(0)
    scat(NSCH - 1, 0)
    wait_a(0)

    plsc.subcore_barrier()
    pltpu.sync_copy(table.at[stripe],
                    z_hbm.at[pl.ds(_m8(c * N + s * NSTRIPE), NSTRIPE)])

    @pl.when(s == 0)
    def _wtail():
        tail = pl.ds(NS_ * NSTRIPE, NTAIL)
        pltpu.sync_copy(table.at[tail],
                        z_hbm.at[pl.ds(_m8(c * N + NS_ * NSTRIPE), NTAIL)])


def _matmul_call(f, n_rows, block_rows, n_in, extra_specs, out_shapes):
    grid = (n_rows // block_rows,)
    return pl.pallas_call(
        f,
        grid=grid,
        in_specs=[pl.BlockSpec((block_rows, n_in), lambda i: (i, 0))] + extra_specs,
        out_specs=[pl.BlockSpec((block_rows, s.shape[1]), lambda i: (i, 0))
                   for s in out_shapes],
        out_shape=list(out_shapes),
    )


def _sds(shape, dtype=jnp.float32):
    return jax.ShapeDtypeStruct(shape, dtype)


def _full(shape):
    return pl.BlockSpec(shape, lambda i: (0, 0))


BN = 1000
BE = 4000


def _node_pre_k(nf, wsd, wn, bn, pqn_o, nc_o):
    x = nf[...]
    pqn_o[...] = jnp.dot(x, wsd[...], preferred_element_type=jnp.float32)
    nc_o[...] = jnp.dot(x, wn[...], preferred_element_type=jnp.float32) + bn[...]


def _edge_k(gsd, hef_in, ef, wa, ba, we, we2, be, hef_o, y_o):
    r = (jnp.dot(hef_in[...], we[...], preferred_element_type=jnp.float32)
         + jnp.dot(ef[...], we2[...], preferred_element_type=jnp.float32)
         + be[...])
    hef = jnp.maximum(gsd[:, :HEF] + r, 0.0)
    hef_o[...] = hef
    logit = jnp.sum(hef * wa[...], axis=1, keepdims=True) + ba[...]
    ex = jnp.exp(logit)
    y_o[...] = jnp.concatenate(
        [hef * ex, ex, jnp.zeros((hef.shape[0], YW - HEF - 1), jnp.float32)],
        axis=1)


def _node_k(z0, z1, hnf, nc, wnh, wna, wsd, pqn, hnf_o, pq_o):
    z = z0[...] + z1[...]
    agg = z[:, :HEF] / (z[:, HEF:HEF + 1] + 1e-16)
    h = (jnp.dot(hnf[...], wnh[...], preferred_element_type=jnp.float32)
         + jnp.dot(agg, wna[...], preferred_element_type=jnp.float32)
         + nc[...])
    h = jnp.maximum(h, 0.0)
    hnf_o[...] = h
    pq_o[...] = jnp.dot(h, wsd[...], preferred_element_type=jnp.float32) + pqn[...]


def _readout_k(x, w, b, o):
    o[...] = jnp.dot(x[...], w[...], preferred_element_type=jnp.float32) + b[...]


def kernel(nf, ef, edge_index, n_iters, W_e, b_e, W_a, b_a, W_n, b_n,
           W_no, b_no, W_eo, b_eo):
    f32 = jnp.float32
    src = edge_index[0]
    dst = edge_index[1]

    We_sd1 = jnp.concatenate([W_e[0:128], W_e[256:384]], axis=1)
    We_sd2 = jnp.concatenate([W_e[128:256], W_e[384:512]], axis=1)
    We_e1 = W_e[512:576]
    We_e2 = W_e[576:592]
    Wn_h = W_n[0:128]
    Wn_nf = W_n[128:256]
    Wn_a = W_n[256:320]
    be = b_e.reshape(1, HEF)
    bn = b_n.reshape(1, HNF)
    wa = W_a.reshape(1, HEF)
    ba = b_a.reshape(1, 1)
    bno = b_no.reshape(1, 128)
    beo = b_eo.reshape(1, HEF)

    pqn, nc = _matmul_call(
        _node_pre_k, N, BN, NF_DIM,
        [_full((NF_DIM, 128)), _full((NF_DIM, HNF)), _full((1, HNF))],
        [_sds((N, 128)), _sds((N, HNF))],
    )(nf, We_sd2, Wn_nf, bn)

    zer = jnp.zeros((N, YW), f32)
    hnf0 = jnp.zeros((N, HNF), f32)

    def make_edge(off_b):
        return pl.pallas_call(
            _edge_k,
            grid=(EH // BE,),
            in_specs=[
                pl.BlockSpec((BE, 128), lambda i: (i, 0)),
                pl.BlockSpec((BE, HEF), lambda i: (i, 0)),
                pl.BlockSpec((BE, 16), lambda i: (i + off_b, 0)),
                _full((1, HEF)), _full((1, 1)), _full((HEF, HEF)),
                _full((16, HEF)), _full((1, HEF))],
            out_specs=[pl.BlockSpec((BE, HEF), lambda i: (i, 0)),
                       pl.BlockSpec((BE, YW), lambda i: (i, 0))],
            out_shape=[_sds((EH, HEF)), _sds((EH, YW))],
        )

    edge_a = make_edge(0)
    edge_b = make_edge(EH // BE)

    node_call = _matmul_call(
        _node_k, N, BN, YW,
        [pl.BlockSpec((BN, YW), lambda i: (i + N // BN, 0)),
         pl.BlockSpec((BN, HNF), lambda i: (i, 0)),
         pl.BlockSpec((BN, HNF), lambda i: (i, 0)),
         _full((HNF, HNF)), _full((HEF, HNF)), _full((HNF, 128)),
         pl.BlockSpec((BN, 128), lambda i: (i, 0))],
        [_sds((N, HNF)), _sds((N, 128))],
    )

    def body(_, carry):
        hnf, hefa, hefb, pq = carry
        gsda = _sc_gather_a(pq, src, dst)
        gsdb = _sc_gather_b(pq, src, dst)
        hef2a, ya = edge_a(gsda, hefa, ef, wa, ba, We_e1, We_e2, be)
        hef2b, yb = edge_b(gsdb, hefb, ef, wa, ba, We_e1, We_e2, be)
        z = _sc_scatter(ya, yb, dst, zer)
        hnf2, pq2 = node_call(z, z, hnf, nc, Wn_h, Wn_a, We_sd1, pqn)
        return (hnf2, hef2a, hef2b, pq2)

    hef0h = jnp.zeros((EH, HEF), f32)
    hnf, hefa, hefb, _ = lax.fori_loop(0, n_iters, body,
                                       (hnf0, hef0h, hef0h, pqn))

    (unf,) = _matmul_call(
        _readout_k, N, BN, HNF,
        [_full((HNF, 128)), _full((1, 128))],
        [_sds((N, 128))],
    )(hnf, W_no, bno)

    (uefa,) = _matmul_call(
        _readout_k, EH, BE, HEF,
        [_full((HEF, HEF)), _full((1, HEF))],
        [_sds((EH, HEF))],
    )(hefa, W_eo, beo)

    (uefb,) = _matmul_call(
        _readout_k, EH, BE, HEF,
        [_full((HEF, HEF)), _full((1, HEF))],
        [_sds((EH, HEF))],
    )(hefb, W_eo, beo)

    uef = jnp.concatenate([uefa, uefb], axis=0)

    return (unf, uef)


# speedup vs baseline: 1.0141x; 1.0141x over previous
import functools

import jax
import jax.numpy as jnp
from jax import lax
from jax.experimental import pallas as pl
from jax.experimental.pallas import tpu as pltpu
from jax.experimental.pallas import tpu_sc as plsc

N = 10000
E = 320000
NF_DIM = 128
HNF = 128
HEF = 64
YW = 128

NC_ = 2
NS_ = 16
NW = NC_ * NS_
EW = E // NW
GCH = 200
NGCH = EW // GCH
NGPAIR = (NGCH - 2) // 2
SCH = 80
SG = 1
NSCH = EW // (SCH * SG)
NSPAIR = (NSCH - 1) // 2
NSTRIPE = 624
NTAIL = N - NS_ * NSTRIPE


def _m8(x):
    return pl.multiple_of(x, 8)


def _m16(x):
    return pl.multiple_of(x, 16)

_mesh = plsc.VectorSubcoreMesh(core_axis_name="c", subcore_axis_name="s")


@functools.partial(
    pl.kernel,
    out_type=jax.ShapeDtypeStruct((E, 128), jnp.float32),
    mesh=_mesh,
    scratch_types=(
        pltpu.VMEM((EW,), jnp.int32),
        pltpu.VMEM((EW,), jnp.int32),
        pltpu.VMEM((2, GCH, 128), jnp.float32),
        pltpu.VMEM((2, GCH, 128), jnp.float32),
        pltpu.SemaphoreType.DMA,
        pltpu.SemaphoreType.DMA,
        pltpu.SemaphoreType.DMA,
        pltpu.SemaphoreType.DMA,
        pltpu.SemaphoreType.DMA,
        pltpu.SemaphoreType.DMA,
    ),
)
def _sc_gather(pq_hbm, src_hbm, dst_hbm, gsd_hbm,
               sidx, didx, rows_s, rows_d, gs0, gs1, gd0, gd1, w0, w1):
    c = lax.axis_index("c")
    s = lax.axis_index("s")
    w = c * NS_ + s
    base0 = w * EW
    pltpu.sync_copy(src_hbm.at[pl.ds(base0, EW)], sidx)
    pltpu.sync_copy(dst_hbm.at[pl.ds(base0, EW)], didx)

    gssem = (gs0, gs1)
    gdsem = (gd0, gd1)
    wsem = (w0, w1)

    def gath(ch, slot):
        pltpu.async_copy(
            pq_hbm.at[sidx.at[pl.ds(ch * GCH, GCH)]], rows_s.at[slot],
            gssem[slot])
        pltpu.async_copy(
            pq_hbm.at[didx.at[pl.ds(ch * GCH, GCH)]], rows_d.at[slot],
            gdsem[slot])

    def wout(ch, slot):
        return pltpu.async_copy(
            rows_s.at[slot], gsd_hbm.at[pl.ds(_m8(base0 + ch * GCH), GCH)],
            wsem[slot])

    def wait_g(slot):
        pltpu.make_async_copy(
            pq_hbm.at[sidx.at[pl.ds(0, GCH)]], rows_s.at[slot],
            gssem[slot]).wait()
        pltpu.make_async_copy(
            pq_hbm.at[didx.at[pl.ds(0, GCH)]], rows_d.at[slot],
            gdsem[slot]).wait()

    def wait_w(slot):
        pltpu.make_async_copy(
            rows_s.at[slot], gsd_hbm.at[pl.ds(base0, GCH)],
            wsem[slot]).wait()

    def add_halves(slot):
        @pl.loop(0, GCH, unroll=2)
        def _add(r):
            for k in range(HEF // 16):
                sl = pl.ds(k * 16, 16)
                sr = pl.ds(HEF + k * 16, 16)
                rows_s[slot, r, sl] = rows_s[slot, r, sl] + rows_d[slot, r, sr]

    gath(0, 0)

    @pl.loop(0, NGPAIR)
    def _pair(ii):
        i0 = 2 * ii

        @pl.when(ii > 0)
        def _():
            wait_w(1)

        gath(i0 + 1, 1)
        wait_g(0)
        add_halves(0)
        wout(i0, 0)
        wait_w(0)
        gath(i0 + 2, 0)
        wait_g(1)
        add_halves(1)
        wout(i0 + 1, 1)

    wait_w(1)
    gath(NGCH - 1, 1)
    wait_g(0)
    add_halves(0)
    wout(NGCH - 2, 0)
    wait_w(0)
    wait_g(1)
    add_halves(1)
    wout(NGCH - 1, 1)
    wait_w(1)


@functools.partial(
    pl.kernel,
    out_type=jax.ShapeDtypeStruct((NC_ * N, YW), jnp.float32),
    mesh=_mesh,
    scratch_types=(
        pltpu.VMEM((2, SCH, YW), jnp.float32),
        [pltpu.VMEM((SCH,), jnp.int32) for _ in range(2)],
        pltpu.VMEM_SHARED((N, YW), jnp.float32),
        pltpu.SemaphoreType.DMA,
        pltpu.SemaphoreType.DMA,
        pltpu.SemaphoreType.DMA,
        pltpu.SemaphoreType.DMA,
        pltpu.SemaphoreType.DMA,
        pltpu.SemaphoreType.DMA,
    ),
)
def _sc_scatter(y_hbm, dst_hbm, zer_hbm, z_hbm,
                ybuf, idxb, table, l0, l1, a0, a1, x0, x1):
    c = lax.axis_index("c")
    s = lax.axis_index("s")
    w = c * NS_ + s
    base0 = w * EW
    stripe = pl.ds(_m8(s * NSTRIPE), NSTRIPE)
    pltpu.sync_copy(zer_hbm.at[stripe], table.at[stripe])

    @pl.when(s == 0)
    def _ztail():
        tail = pl.ds(NS_ * NSTRIPE, NTAIL)
        pltpu.sync_copy(zer_hbm.at[tail], table.at[tail])

    lsem = (l0, l1)
    asem = (a0, a1)
    xsem = (x0, x1)

    def load(ch, slot):
        pltpu.async_copy(
            y_hbm.at[pl.ds(_m8(base0 + ch * SCH), SCH)],
            ybuf.at[slot], lsem[slot])
        pltpu.async_copy(
            dst_hbm.at[pl.ds(_m8(base0 + ch * SCH), SCH)],
            idxb[slot], xsem[slot])

    def scat(ch, slot):
        pltpu.async_copy(
            ybuf.at[slot], table.at[idxb[slot]], asem[slot], add=True)

    def wait_l(slot):
        pltpu.make_async_copy(
            y_hbm.at[pl.ds(base0, SCH)], ybuf.at[slot], lsem[slot]).wait()
        pltpu.make_async_copy(
            dst_hbm.at[pl.ds(base0, SCH)], idxb[slot], xsem[slot]).wait()

    def wait_a(slot):
        pltpu.make_async_copy(
            ybuf.at[slot], table.at[idxb[slot]], asem[slot]).wait()

    load(0, 0)
    plsc.subcore_barrier()

    @pl.loop(0, NSPAIR)
    def _pair(ii):
        i0 = 2 * ii

        @pl.when(ii > 0)
        def _():
            wait_a(1)

        load(i0 + 1, 1)
        wait_l(0)
        scat(i0, 0)
        wait_a(0)
        load(i0 + 2, 0)
        wait_l(1)
        scat(i0 + 1, 1)

    wait_a(1)
    wait_l(0)
    scat(NSCH - 1, 0)
    wait_a(0)

    plsc.subcore_barrier()
    pltpu.sync_copy(table.at[stripe],
                    z_hbm.at[pl.ds(_m8(c * N + s * NSTRIPE), NSTRIPE)])

    @pl.when(s == 0)
    def _wtail():
        tail = pl.ds(NS_ * NSTRIPE, NTAIL)
        pltpu.sync_copy(table.at[tail],
                        z_hbm.at[pl.ds(_m8(c * N + NS_ * NSTRIPE), NTAIL)])


def _matmul_call(f, n_rows, block_rows, n_in, extra_specs, out_shapes):
    grid = (n_rows // block_rows,)
    return pl.pallas_call(
        f,
        grid=grid,
        in_specs=[pl.BlockSpec((block_rows, n_in), lambda i: (i, 0))] + extra_specs,
        out_specs=[pl.BlockSpec((block_rows, s.shape[1]), lambda i: (i, 0))
                   for s in out_shapes],
        out_shape=list(out_shapes),
    )


def _sds(shape, dtype=jnp.float32):
    return jax.ShapeDtypeStruct(shape, dtype)


def _full(shape):
    return pl.BlockSpec(shape, lambda i: (0, 0))


BN = 1000
BE = 4000


def _node_pre_k(nf, wsd, wn, bn, pqn_o, nc_o):
    x = nf[...]
    pqn_o[...] = jnp.dot(x, wsd[...], preferred_element_type=jnp.float32)
    nc_o[...] = jnp.dot(x, wn[...], preferred_element_type=jnp.float32) + bn[...]


def _edge_k(gsd, hef_in, ef, wa, ba, we, we2, be, hef_o, y_o):
    r = (jnp.dot(hef_in[...], we[...], preferred_element_type=jnp.float32)
         + jnp.dot(ef[...], we2[...], preferred_element_type=jnp.float32)
         + be[...])
    hef = jnp.maximum(gsd[:, :HEF] + r, 0.0)
    hef_o[...] = hef
    logit = jnp.sum(hef * wa[...], axis=1, keepdims=True) + ba[...]
    ex = jnp.exp(logit)
    y_o[...] = jnp.concatenate(
        [hef * ex, ex, jnp.zeros((hef.shape[0], YW - HEF - 1), jnp.float32)],
        axis=1)


def _node_k(z0, z1, hnf, nc, wnh, wna, wsd, pqn, hnf_o, pq_o):
    z = z0[...] + z1[...]
    agg = z[:, :HEF] / (z[:, HEF:HEF + 1] + 1e-16)
    h = (jnp.dot(hnf[...], wnh[...], preferred_element_type=jnp.float32)
         + jnp.dot(agg, wna[...], preferred_element_type=jnp.float32)
         + nc[...])
    h = jnp.maximum(h, 0.0)
    hnf_o[...] = h
    pq_o[...] = jnp.dot(h, wsd[...], preferred_element_type=jnp.float32) + pqn[...]


def _readout_k(x, w, b, o):
    o[...] = jnp.dot(x[...], w[...], preferred_element_type=jnp.float32) + b[...]


def kernel(nf, ef, edge_index, n_iters, W_e, b_e, W_a, b_a, W_n, b_n,
           W_no, b_no, W_eo, b_eo):
    f32 = jnp.float32
    src = edge_index[0]
    dst = edge_index[1]

    We_sd1 = jnp.concatenate([W_e[0:128], W_e[256:384]], axis=1)
    We_sd2 = jnp.concatenate([W_e[128:256], W_e[384:512]], axis=1)
    We_e1 = W_e[512:576]
    We_e2 = W_e[576:592]
    Wn_h = W_n[0:128]
    Wn_nf = W_n[128:256]
    Wn_a = W_n[256:320]
    be = b_e.reshape(1, HEF)
    bn = b_n.reshape(1, HNF)
    wa = W_a.reshape(1, HEF)
    ba = b_a.reshape(1, 1)
    bno = b_no.reshape(1, 128)
    beo = b_eo.reshape(1, HEF)

    pqn, nc = _matmul_call(
        _node_pre_k, N, BN, NF_DIM,
        [_full((NF_DIM, 128)), _full((NF_DIM, HNF)), _full((1, HNF))],
        [_sds((N, 128)), _sds((N, HNF))],
    )(nf, We_sd2, Wn_nf, bn)

    zer = jnp.zeros((N, YW), f32)
    hnf0 = jnp.zeros((N, HNF), f32)

    edge_call = pl.pallas_call(
        _edge_k,
        grid=(E // BE,),
        in_specs=[
            pl.BlockSpec((BE, 128), lambda i: (i, 0)),
            pl.BlockSpec((BE, HEF), lambda i: (i, 0)),
            pl.BlockSpec((BE, 16), lambda i: (i, 0)),
            _full((1, HEF)), _full((1, 1)), _full((HEF, HEF)),
            _full((16, HEF)), _full((1, HEF))],
        out_specs=[pl.BlockSpec((BE, HEF), lambda i: (i, 0)),
                   pl.BlockSpec((BE, YW), lambda i: (i, 0))],
        out_shape=[_sds((E, HEF)), _sds((E, YW))],
    )

    node_call = _matmul_call(
        _node_k, N, BN, YW,
        [pl.BlockSpec((BN, YW), lambda i: (i + N // BN, 0)),
         pl.BlockSpec((BN, HNF), lambda i: (i, 0)),
         pl.BlockSpec((BN, HNF), lambda i: (i, 0)),
         _full((HNF, HNF)), _full((HEF, HNF)), _full((HNF, 128)),
         pl.BlockSpec((BN, 128), lambda i: (i, 0))],
        [_sds((N, HNF)), _sds((N, 128))],
    )

    def body(_, carry):
        hnf, hef, pq = carry
        gsd = _sc_gather(pq, src, dst)
        hef2, y = edge_call(gsd, hef, ef, wa, ba, We_e1, We_e2, be)
        z = _sc_scatter(y, dst, zer)
        hnf2, pq2 = node_call(z, z, hnf, nc, Wn_h, Wn_a, We_sd1, pqn)
        return (hnf2, hef2, pq2)

    hef0 = jnp.zeros((E, HEF), f32)
    hnf, hef, _ = lax.fori_loop(0, n_iters, body, (hnf0, hef0, pqn))

    (unf,) = _matmul_call(
        _readout_k, N, BN, HNF,
        [_full((HNF, 128)), _full((1, 128))],
        [_sds((N, 128))],
    )(hnf, W_no, bno)

    (uef,) = _matmul_call(
        _readout_k, E, BE, HEF,
        [_full((HEF, HEF)), _full((1, HEF))],
        [_sds((E, HEF))],
    )(hef, W_eo, beo)

    return (unf, uef)
